# baseline (speedup 1.0000x reference)
import functools

import jax
import jax.numpy as jnp
import numpy as np
from jax import lax
from jax.experimental import pallas as pl
from jax.experimental.pallas import tpu as pltpu
from jax.experimental.pallas import tpu_sc as plsc

_BATCH = 4096
_NROWS = 100000
_DIM = 128
_NW = 32
_RPW = _BATCH // _NW
_NROLE = 16
_CHUNK = 512


def _tf2x32(k1, k2, x0, x1):
    rotations = ((13, 15, 26, 6), (17, 29, 16, 24))
    ks = (np.uint32(k1), np.uint32(k2),
          np.uint32(k1) ^ np.uint32(k2) ^ np.uint32(0x1BD11BDA))
    x0 = x0.astype(np.uint32) + ks[0]
    x1 = x1.astype(np.uint32) + ks[1]
    with np.errstate(over="ignore"):
        for d in range(5):
            for r in rotations[d % 2]:
                x0 = x0 + x1
                x1 = (x1 << np.uint32(r)) | (x1 >> np.uint32(32 - r))
                x1 = x1 ^ x0
            x0 = x0 + ks[(d + 1) % 3]
            x1 = x1 + ks[(d + 2) % 3] + np.uint32(d + 1)
    return x0, x1


def _sample_indices_np(seed, batch, maxval):
    f0, f1 = _tf2x32(np.uint32(0), np.uint32(1),
                     np.uint32([0]), np.uint32([seed]))
    s0, s1 = _tf2x32(f0[0], f1[0], np.uint32([0, 0]), np.uint32([0, 1]))
    ar = np.arange(batch, dtype=np.uint32)
    zr = np.zeros(batch, dtype=np.uint32)
    o0, o1 = _tf2x32(s0[0], s1[0], zr, ar)
    y = o0 ^ o1
    o0, o1 = _tf2x32(s0[1], s1[1], zr, ar)
    z = o0 ^ o1
    span = np.uint32(maxval)
    with np.errstate(over="ignore"):
        mult = (np.uint32(65536 % maxval) * np.uint32(65536 % maxval)) % span
        b = ((y % span) * mult + (z % span)) % span
    return b.astype(np.int32)


@functools.lru_cache(maxsize=None)
def _flat_offsets():
    s = [_sample_indices_np(i, _BATCH, _NROWS) for i in range(6)]
    rows = [
        s[0] * 2 + 0, s[0] * 2 + 1,
        s[1] * 3 + 0, s[1] * 3 + 1, s[1] * 3 + 2,
        s[2] * 3 + 0, s[2] * 3 + 1, s[2] * 3 + 2,
        s[3] * 3 + 0, s[3] * 3 + 1, s[3] * 3 + 2,
        s[4] * 2 + 0, s[4] * 2 + 1,
        s[5] * 3 + 0, s[5] * 3 + 1, s[5] * 3 + 2,
    ]
    return np.stack(rows).astype(np.int32)


_ROLES = (
    (0, True), (0, True),
    (1, True), (1, True), (1, True),
    (2, True), (2, False), (2, True),
    (3, False), (3, True), (3, True),
    (4, True), (4, True),
    (5, True), (5, False), (5, True),
)


def _sc_body(xs_h, rad_h, rel_h, nf1_h, nf2_h, nf3_h, nf4_h, dj_h, neg_h,
             fidx_h, rows_h, radout_h,
             fv, cid, radv, buf0, buf1, buf2,
             sem_i0, sem_i1, sem_i2, sem_o0, sem_o1, sem_o2, sem_r):
    cidx = lax.axis_index("c")
    sidx = lax.axis_index("s")
    wid = sidx * 2 + cidx
    base = wid * _RPW
    nf_tabs = (nf1_h, nf2_h, nf3_h, nf4_h, dj_h, neg_h)
    bufs = (buf0, buf1, buf2)
    sem_in = (sem_i0, sem_i1, sem_i2)
    sem_out = (sem_o0, sem_o1, sem_o2)

    for r in range(_NROLE):
        pltpu.sync_copy(fidx_h.at[r, pl.ds(base, _RPW)],
                        fv.at[r, pl.ds(0, _RPW)])
    id_descs = [None] * _NROLE
    for r, (tab, _) in enumerate(_ROLES):
        if r >= 3:
            id_descs[r - 3].wait()
        id_descs[r] = pltpu.async_copy(
            nf_tabs[tab].at[fv.at[r, pl.ds(0, _RPW)]],
            cid.at[r, pl.ds(0, _RPW)], sem_in[r % 3])
    for r in range(_NROLE - 3, _NROLE):
        id_descs[r].wait()

    out_descs = [None] * _NROLE
    rad_descs = []
    for r, (_, is_class) in enumerate(_ROLES):
        p = r % 3
        if r >= 3:
            out_descs[r - 3].wait()
        tab = xs_h if is_class else rel_h
        g = pltpu.async_copy(tab.at[cid.at[r, pl.ds(0, _RPW)]], bufs[p],
                             sem_in[p])
        if is_class:
            rad_descs.append(pltpu.async_copy(
                rad_h.at[cid.at[r, pl.ds(0, _RPW)]],
                radv.at[r, pl.ds(0, _RPW)], sem_r))
        g.wait()
        out_descs[r] = pltpu.async_copy(
            bufs[p], rows_h.at[r, pl.ds(base, _RPW)], sem_out[p])
    for r in range(_NROLE - 3, _NROLE):
        out_descs[r].wait()
    for d in rad_descs:
        d.wait()
    pltpu.async_copy(radv, radout_h.at[pl.ds(0, _NROLE), pl.ds(base, _RPW)],
                     sem_r).wait()


def _tc_body(rows_ref, rad_ref, out_ref):
    relu = jax.nn.relu

    def rows(r):
        return rows_ref[r]

    def srad(r):
        return jnp.abs(rad_ref[r, :])

    def sq(x):
        return jnp.sum(x * x, axis=-1)

    def reg(x):
        return jnp.abs(jnp.sqrt(sq(x)) - 1.0)

    total = jnp.float32(0.0)

    a, b = rows(0), rows(1)
    ra, rb = srad(0), srad(1)
    e = relu(jnp.abs(a - b) + (ra - rb)[:, None])
    total += jnp.sum(jnp.sum(e, axis=-1) / _DIM + reg(a) + reg(b))

    a, b, c = rows(2), rows(3), rows(4)
    ra, rb, rc = srad(2), srad(3), srad(4)
    t = (relu(jnp.sqrt(sq(b - a)) - (ra + rb))
         + relu(jnp.sqrt(sq(c - a)) - ra)
         + relu(jnp.sqrt(sq(c - b)) - rb)
         + relu(jnp.minimum(ra, rb) - rc)
         + reg(a) + reg(b) + reg(c))
    total += jnp.sum(t)

    a, r_, b = rows(5), rows(6), rows(7)
    ra, rb = srad(5), srad(7)
    total += jnp.sum(relu(jnp.sqrt(sq(a + r_ - b)) + ra - rb)
                     + reg(a) + reg(b))

    r_, a, b = rows(8), rows(9), rows(10)
    ra, rb = srad(9), srad(10)
    total += jnp.sum(relu(jnp.sqrt(sq(a - r_ - b)) - (ra + rb))
                     + reg(a) + reg(b))

    a, b = rows(11), rows(12)
    ra, rb = srad(11), srad(12)
    total += jnp.sum(relu(ra + rb - jnp.sqrt(sq(b - a))) + reg(a) + reg(b))

    a, r_, b = rows(13), rows(14), rows(15)
    ra, rb = srad(13), srad(15)
    total += jnp.sum((ra + rb - jnp.sqrt(sq(a + r_ - b)))
                     + reg(a) + reg(b))

    @pl.when(pl.program_id(0) == 0)
    def _init():
        out_ref[0, 0] = 0.0

    out_ref[0, 0] += total / _BATCH


def kernel(class_emb, rel_emb, nf1, nf2, nf3, nf4, disjoint, nf3_neg):
    class_emb = class_emb.astype(jnp.float32)
    xs = class_emb[:, :_DIM]
    rad = class_emb[:, _DIM]
    rel = rel_emb.astype(jnp.float32)
    nfs = [a.astype(jnp.int32).reshape(-1)
           for a in (nf1, nf2, nf3, nf4, disjoint, nf3_neg)]
    fidx = jnp.asarray(_flat_offsets())

    mesh = plsc.VectorSubcoreMesh(
        core_axis_name="c", subcore_axis_name="s", num_cores=2,
        num_subcores=16)
    sc_run = pl.kernel(
        _sc_body,
        out_type=[
            jax.ShapeDtypeStruct((_NROLE, _BATCH, _DIM), jnp.float32),
            jax.ShapeDtypeStruct((_NROLE, _BATCH), jnp.float32),
        ],
        mesh=mesh,
        scratch_types=[
            pltpu.VMEM((_NROLE, _RPW), jnp.int32),
            pltpu.VMEM((_NROLE, _RPW), jnp.int32),
            pltpu.VMEM((_NROLE, _RPW), jnp.float32),
            pltpu.VMEM((_RPW, _DIM), jnp.float32),
            pltpu.VMEM((_RPW, _DIM), jnp.float32),
            pltpu.VMEM((_RPW, _DIM), jnp.float32),
            pltpu.SemaphoreType.DMA,
            pltpu.SemaphoreType.DMA,
            pltpu.SemaphoreType.DMA,
            pltpu.SemaphoreType.DMA,
            pltpu.SemaphoreType.DMA,
            pltpu.SemaphoreType.DMA,
            pltpu.SemaphoreType.DMA,
        ],
    )
    rows, rads = sc_run(xs, rad, rel, *nfs, fidx)

    total = pl.pallas_call(
        _tc_body,
        grid=(_BATCH // _CHUNK,),
        in_specs=[
            pl.BlockSpec((_NROLE, _CHUNK, _DIM), lambda i: (0, i, 0)),
            pl.BlockSpec((_NROLE, _CHUNK), lambda i: (0, i)),
        ],
        out_shape=jax.ShapeDtypeStruct((1, 1), jnp.float32),
        out_specs=pl.BlockSpec((1, 1), lambda i: (0, 0),
                               memory_space=pltpu.SMEM),
    )(rows, rads)
    return total[0, 0]


# speedup vs baseline: 1.1020x; 1.1020x over previous
import functools

import jax
import jax.numpy as jnp
import numpy as np
from jax import lax
from jax.experimental import pallas as pl
from jax.experimental.pallas import tpu as pltpu
from jax.experimental.pallas import tpu_sc as plsc

_BATCH = 4096
_NROWS = 100000
_DIM = 128
_NW = 32
_RPW = _BATCH // _NW
_NROLE = 16
_PAD = 1024
_NSV = 45


def _tf2x32(k1, k2, x0, x1):
    rotations = ((13, 15, 26, 6), (17, 29, 16, 24))
    ks = (np.uint32(k1), np.uint32(k2),
          np.uint32(k1) ^ np.uint32(k2) ^ np.uint32(0x1BD11BDA))
    x0 = x0.astype(np.uint32) + ks[0]
    x1 = x1.astype(np.uint32) + ks[1]
    with np.errstate(over="ignore"):
        for d in range(5):
            for r in rotations[d % 2]:
                x0 = x0 + x1
                x1 = (x1 << np.uint32(r)) | (x1 >> np.uint32(32 - r))
                x1 = x1 ^ x0
            x0 = x0 + ks[(d + 1) % 3]
            x1 = x1 + ks[(d + 2) % 3] + np.uint32(d + 1)
    return x0, x1


def _sample_indices_np(seed, batch, maxval):
    f0, f1 = _tf2x32(np.uint32(0), np.uint32(1),
                     np.uint32([0]), np.uint32([seed]))
    s0, s1 = _tf2x32(f0[0], f1[0], np.uint32([0, 0]), np.uint32([0, 1]))
    ar = np.arange(batch, dtype=np.uint32)
    zr = np.zeros(batch, dtype=np.uint32)
    o0, o1 = _tf2x32(s0[0], s1[0], zr, ar)
    y = o0 ^ o1
    o0, o1 = _tf2x32(s0[1], s1[1], zr, ar)
    z = o0 ^ o1
    span = np.uint32(maxval)
    with np.errstate(over="ignore"):
        mult = (np.uint32(65536 % maxval) * np.uint32(65536 % maxval)) % span
        b = ((y % span) * mult + (z % span)) % span
    return b.astype(np.int32)


@functools.lru_cache(maxsize=None)
def _flat_offsets():
    s = [_sample_indices_np(i, _BATCH, _NROWS) for i in range(6)]
    rows = [
        s[0] * 2 + 0, s[0] * 2 + 1,
        s[1] * 3 + 0, s[1] * 3 + 1, s[1] * 3 + 2,
        s[2] * 3 + 0, s[2] * 3 + 1, s[2] * 3 + 2,
        s[3] * 3 + 0, s[3] * 3 + 1, s[3] * 3 + 2,
        s[4] * 2 + 0, s[4] * 2 + 1,
        s[5] * 3 + 0, s[5] * 3 + 1, s[5] * 3 + 2,
    ]
    return np.stack(rows).astype(np.int32)


_ROLE_TAB = (0, 0, 1, 1, 1, 2, 2, 2, 3, 3, 3, 4, 4, 5, 5, 5)
_CLASS_ROLES = (0, 1, 2, 3, 4, 5, 7, 9, 10, 11, 12, 13, 15)
_REL_ROLES = (6, 8, 14)
_COMBOS = (
    (32, "G", 2, 3), (33, "G", 2, 4), (34, "G", 3, 4),
    (35, "G", 5, 7), (36, "C", 5, 6), (37, "C", 7, 6),
    (38, "G", 9, 10), (39, "C", 9, 8), (40, "C", 10, 8),
    (41, "G", 11, 12),
    (42, "G", 13, 15), (43, "C", 13, 14), (44, "C", 15, 14),
)


def _tc1_body(xs_ref, xst_ref, relt_ref, g_ref, c_ref, nr_ref):
    x = xs_ref[...]
    g_ref[...] = jnp.dot(x, xst_ref[...], preferred_element_type=jnp.float32)
    c_ref[...] = jnp.dot(x, relt_ref[...], preferred_element_type=jnp.float32)
    rt = relt_ref[...]
    nr_ref[...] = jnp.sum(rt * rt, axis=0, keepdims=True)


def _sc_body(g_h, c_h, nr_h, rad_h, xs_h,
             nf1_h, nf2_h, nf3_h, nf4_h, dj_h, neg_h, fidx_h,
             sv_h, rows_h,
             fv, cid, gidx, sv, buf0, buf1,
             s0, s1, s2, s3, s4, s5, s6):
    cidx = lax.axis_index("c")
    sidx = lax.axis_index("s")
    wid = sidx * 2 + cidx
    base = wid * _RPW
    nf_tabs = (nf1_h, nf2_h, nf3_h, nf4_h, dj_h, neg_h)
    sems = (s0, s1, s2, s3)

    pltpu.async_copy(fidx_h.at[pl.ds(0, _NROLE), pl.ds(base, _RPW)],
                     fv, s4).wait()

    descs = []
    for r in range(_NROLE):
        if r >= 4:
            descs[r - 4].wait()
        descs.append(pltpu.async_copy(
            nf_tabs[_ROLE_TAB[r]].at[fv.at[r, pl.ds(0, _RPW)]],
            cid.at[r, pl.ds(0, _RPW)], sems[r % 4]))
    for d in descs[-4:]:
        d.wait()

    rg0 = pltpu.async_copy(xs_h.at[cid.at[0, pl.ds(0, _RPW)]], buf0, s4)
    rg1 = pltpu.async_copy(xs_h.at[cid.at[1, pl.ds(0, _RPW)]], buf1, s5)

    for n, role in enumerate(_CLASS_ROLES):
        for k in range(_RPW // 16):
            v = cid[role, pl.ds(k * 16, 16)]
            gidx[n, pl.ds(k * 16, 16)] = v * 1025
    for n, (_, _, ri, rj) in enumerate(_COMBOS):
        for k in range(_RPW // 16):
            vi = cid[ri, pl.ds(k * 16, 16)]
            vj = cid[rj, pl.ds(k * 16, 16)]
            gidx[13 + n, pl.ds(k * 16, 16)] = vi * 1024 + vj

    gd = []

    def q(table, idx_ref, slot):
        if len(gd) >= 4:
            gd[len(gd) - 4].wait()
        gd.append(pltpu.async_copy(table.at[idx_ref],
                                   sv.at[slot, pl.ds(0, _RPW)],
                                   sems[len(gd) % 4]))

    for n, role in enumerate(_CLASS_ROLES):
        q(g_h, gidx.at[n, pl.ds(0, _RPW)], role)
        q(rad_h, cid.at[role, pl.ds(0, _RPW)], 16 + role)
    for role in _REL_ROLES:
        q(nr_h, cid.at[role, pl.ds(0, _RPW)], role)
    for n, (slot, tab, _, _) in enumerate(_COMBOS):
        q(g_h if tab == "G" else c_h,
          gidx.at[13 + n, pl.ds(0, _RPW)], slot)
    for d in gd[-4:]:
        d.wait()

    rg0.wait()
    rg1.wait()
    o0 = pltpu.async_copy(buf0, rows_h.at[0, pl.ds(base, _RPW)], s4)
    o1 = pltpu.async_copy(buf1, rows_h.at[1, pl.ds(base, _RPW)], s5)
    o2 = pltpu.async_copy(sv, sv_h.at[pl.ds(0, _NSV), pl.ds(base, _RPW)], s6)
    o0.wait()
    o1.wait()
    o2.wait()


def _tc2_body(sv_ref, rows_ref, out_ref):
    relu = jax.nn.relu

    def sv(i):
        return sv_ref[i, :]

    def rad(role):
        return jnp.abs(sv(16 + role))

    def reg(n):
        return jnp.abs(jnp.sqrt(n) - 1.0)

    def dist(arg):
        return jnp.sqrt(jnp.maximum(arg, 0.0))

    total = jnp.float32(0.0)

    a, b = rows_ref[0], rows_ref[1]
    ra, rb = rad(0), rad(1)
    e = relu(jnp.abs(a - b) + (ra - rb)[:, None])
    total += jnp.sum(jnp.sum(e, axis=-1) / _DIM + reg(sv(0)) + reg(sv(1)))

    na, nb, nc = sv(2), sv(3), sv(4)
    ra, rb, rc = rad(2), rad(3), rad(4)
    dab = dist(na + nb - 2.0 * sv(32))
    dac = dist(na + nc - 2.0 * sv(33))
    dbc = dist(nb + nc - 2.0 * sv(34))
    total += jnp.sum(relu(dab - (ra + rb)) + relu(dac - ra)
                     + relu(dbc - rb) + relu(jnp.minimum(ra, rb) - rc)
                     + reg(na) + reg(nb) + reg(nc))

    na, nb, nr = sv(5), sv(7), sv(6)
    ra, rb = rad(5), rad(7)
    euc = dist(na + nb + nr - 2.0 * sv(35) + 2.0 * sv(36) - 2.0 * sv(37))
    total += jnp.sum(relu(euc + ra - rb) + reg(na) + reg(nb))

    na, nb, nr = sv(9), sv(10), sv(8)
    ra, rb = rad(9), rad(10)
    euc = dist(na + nb + nr - 2.0 * sv(38) - 2.0 * sv(39) + 2.0 * sv(40))
    total += jnp.sum(relu(euc - (ra + rb)) + reg(na) + reg(nb))

    na, nb = sv(11), sv(12)
    ra, rb = rad(11), rad(12)
    euc = dist(na + nb - 2.0 * sv(41))
    total += jnp.sum(relu(ra + rb - euc) + reg(na) + reg(nb))

    na, nb, nr = sv(13), sv(15), sv(14)
    ra, rb = rad(13), rad(15)
    euc = dist(na + nb + nr - 2.0 * sv(42) + 2.0 * sv(43) - 2.0 * sv(44))
    total += jnp.sum((ra + rb - euc) + reg(na) + reg(nb))

    out_ref[0, 0] = total / _BATCH


def kernel(class_emb, rel_emb, nf1, nf2, nf3, nf4, disjoint, nf3_neg):
    class_emb = class_emb.astype(jnp.float32)
    xs = class_emb[:, :_DIM]
    rad = class_emb[:, _DIM]
    rel = rel_emb.astype(jnp.float32)
    nfs = [a.astype(jnp.int32).reshape(-1)
           for a in (nf1, nf2, nf3, nf4, disjoint, nf3_neg)]
    fidx = jnp.asarray(_flat_offsets())

    pad = jnp.zeros((_PAD - xs.shape[0], _DIM), jnp.float32)
    xs_p = jnp.concatenate([xs, pad], axis=0)
    rel_p = jnp.concatenate([rel, pad], axis=0)

    gram, cross, nrm = pl.pallas_call(
        _tc1_body,
        out_shape=[
            jax.ShapeDtypeStruct((_PAD, _PAD), jnp.float32),
            jax.ShapeDtypeStruct((_PAD, _PAD), jnp.float32),
            jax.ShapeDtypeStruct((1, _PAD), jnp.float32),
        ],
    )(xs_p, xs_p.T, rel_p.T)

    mesh = plsc.VectorSubcoreMesh(
        core_axis_name="c", subcore_axis_name="s", num_cores=2,
        num_subcores=16)
    sc_run = pl.kernel(
        _sc_body,
        out_type=[
            jax.ShapeDtypeStruct((_NSV, _BATCH), jnp.float32),
            jax.ShapeDtypeStruct((2, _BATCH, _DIM), jnp.float32),
        ],
        mesh=mesh,
        scratch_types=[
            pltpu.VMEM((_NROLE, _RPW), jnp.int32),
            pltpu.VMEM((_NROLE, _RPW), jnp.int32),
            pltpu.VMEM((26, _RPW), jnp.int32),
            pltpu.VMEM((_NSV, _RPW), jnp.float32),
            pltpu.VMEM((_RPW, _DIM), jnp.float32),
            pltpu.VMEM((_RPW, _DIM), jnp.float32),
            pltpu.SemaphoreType.DMA,
            pltpu.SemaphoreType.DMA,
            pltpu.SemaphoreType.DMA,
            pltpu.SemaphoreType.DMA,
            pltpu.SemaphoreType.DMA,
            pltpu.SemaphoreType.DMA,
            pltpu.SemaphoreType.DMA,
        ],
    )
    svals, rows = sc_run(gram.reshape(-1), cross.reshape(-1),
                         nrm.reshape(-1), rad, xs, *nfs, fidx)

    total = pl.pallas_call(
        _tc2_body,
        out_shape=jax.ShapeDtypeStruct((1, 1), jnp.float32),
        out_specs=pl.BlockSpec(memory_space=pltpu.SMEM),
    )(svals, rows)
    return total[0, 0]
